# baseline (speedup 1.0000x reference)
import functools

import jax
import jax.numpy as jnp
from jax import lax
from jax.experimental import pallas as pl
from jax.experimental.pallas import tpu as pltpu
from jax.experimental.pallas import tpu_sc as plsc

NC = 2
NS = 16
L = 16
NW = NC * NS

CHUNK = 128

CORE0_SHARE = 0.84


def _sc_mesh():
    return plsc.VectorSubcoreMesh(
        core_axis_name="c", subcore_axis_name="s", num_cores=NC, num_subcores=NS
    )


def _sc_agg_body(k0, k1, ib, stripe, table_hbm, src_hbm, dst_hbm, zeros_hbm,
                 agg_out, src_v, dst_v, rows0_v, rows1_v, agg_sh,
                 gsem, ssem0, ssem1):
    c = lax.axis_index("c")
    s = lax.axis_index("s")
    w = c * NS + s
    row0 = s * stripe
    k_eff = jnp.where(c == 0, k0, k1)

    pltpu.sync_copy(zeros_hbm.at[pl.ds(row0, stripe)],
                    agg_sh.at[pl.ds(row0, stripe)])
    plsc.subcore_barrier()

    def block(b, carry):
        pltpu.sync_copy(src_hbm.at[w, pl.ds(b * ib, ib)], src_v)
        pltpu.sync_copy(dst_hbm.at[w, pl.ds(b * ib, ib)], dst_v)
        pairs = jnp.clip((k_eff - b * ib) // 2, 0, ib // 2)

        def pair(m, carry2):
            j0 = 2 * m
            j1 = 2 * m + 1
            g0 = 2 * m + b * ib
            g1 = g0 + 1
            pltpu.async_copy(table_hbm.at[src_v.at[j0]], rows0_v,
                             gsem).wait()

            @pl.when(m > 0)
            def _():
                pltpu.make_async_copy(
                    rows1_v, agg_sh.at[dst_v.at[j1 - 2]], ssem1).wait()

            pltpu.async_copy(rows0_v, agg_sh.at[dst_v.at[j0]], ssem0,
                             add=True)
            pltpu.async_copy(table_hbm.at[src_v.at[j1]], rows1_v,
                             gsem).wait()
            pltpu.make_async_copy(rows0_v, agg_sh.at[dst_v.at[j0]],
                                  ssem0).wait()
            pltpu.async_copy(rows1_v, agg_sh.at[dst_v.at[j1]], ssem1,
                             add=True)
            return carry2

        lax.fori_loop(0, pairs, pair, 0)

        @pl.when(pairs > 0)
        def _():
            last = 2 * pairs - 1
            pltpu.make_async_copy(rows1_v, agg_sh.at[dst_v.at[last]],
                                  ssem1).wait()
        return carry

    lax.fori_loop(0, (jnp.maximum(k_eff, 1) + ib - 1) // ib, block, 0)
    plsc.subcore_barrier()

    pltpu.sync_copy(agg_sh.at[pl.ds(row0, stripe)],
                    agg_out.at[c, pl.ds(row0, stripe)])


def _make_sc_agg(n_pad, k0, k1, ib, d):
    stripe = n_pad // NS
    return pl.kernel(
        functools.partial(_sc_agg_body, k0, k1, ib, stripe),
        out_type=jax.ShapeDtypeStruct((NC, n_pad, d), jnp.float32),
        mesh=_sc_mesh(),
        scratch_types=[
            pltpu.VMEM((ib, CHUNK), jnp.int32),
            pltpu.VMEM((ib, CHUNK), jnp.int32),
            pltpu.VMEM((CHUNK, d), jnp.float32),
            pltpu.VMEM((CHUNK, d), jnp.float32),
            pltpu.VMEM_SHARED((n_pad, d), jnp.float32),
            pltpu.SemaphoreType.DMA,
            pltpu.SemaphoreType.DMA,
            pltpu.SemaphoreType.DMA,
        ],
        name="sc_segment_sum",
    )


def _sc_cnt_body(k0, k1, stripe, d, dst_hbm, zeros_hbm, ones_hbm,
                 cnt_out, dst_v, ones_v, cnt_sh):
    c = lax.axis_index("c")
    s = lax.axis_index("s")
    w = c * NS + s
    row0 = s * stripe

    pltpu.sync_copy(dst_hbm.at[w], dst_v)
    pltpu.sync_copy(ones_hbm, ones_v)
    pltpu.sync_copy(zeros_hbm.at[pl.ds(row0, stripe)],
                    cnt_sh.at[pl.ds(row0, stripe)])
    plsc.subcore_barrier()

    def step(j, carry):
        pltpu.sync_copy(ones_v, cnt_sh.at[dst_v.at[j]], add=True)
        return carry

    lax.fori_loop(0, jnp.where(c == 0, k0, k1), step, 0)
    plsc.subcore_barrier()
    pltpu.sync_copy(cnt_sh.at[pl.ds(row0, stripe)],
                    cnt_out.at[c, pl.ds(row0, stripe)])


def _make_sc_cnt(n_pad, k0, k1, d):
    stripe = n_pad // NS
    return pl.kernel(
        functools.partial(_sc_cnt_body, k0, k1, stripe, d),
        out_type=jax.ShapeDtypeStruct((NC, n_pad, d), jnp.float32),
        mesh=_sc_mesh(),
        scratch_types=[
            pltpu.VMEM((max(k0, k1), CHUNK), jnp.int32),
            pltpu.VMEM((CHUNK, d), jnp.float32),
            pltpu.VMEM_SHARED((n_pad, d), jnp.float32),
        ],
        name="sc_degree_count",
    )


def _sc_pairs_body(pb, *refs):
    (u_hbm, v_hbm, p0_hbm, p1_hbm, out_hbm,
     u_v, v_v, p0_v, p1_v, out_v) = refs
    c = lax.axis_index("c")
    s = lax.axis_index("s")
    w = c * NS + s

    pltpu.sync_copy(u_hbm, u_v)
    pltpu.sync_copy(v_hbm, v_v)
    pltpu.sync_copy(p0_hbm.at[w], p0_v)
    pltpu.sync_copy(p1_hbm.at[w], p1_v)

    def step(i, carry):
        i0 = p0_v[pl.ds(i * L, L)]
        i1 = p1_v[pl.ds(i * L, L)]
        a = plsc.load_gather(u_v, [lax.shift_right_logical(i0, 7),
                                   lax.bitwise_and(i0, 127)])
        b = plsc.load_gather(v_v, [lax.shift_right_logical(i1, 7),
                                   lax.bitwise_and(i1, 127)])
        z = a + b
        out_v[pl.ds(i * L, L)] = 1.0 / (1.0 + jnp.exp(-z))
        return carry

    lax.fori_loop(0, pb // L, step, 0)
    pltpu.sync_copy(out_v, out_hbm.at[w])


def _make_sc_pairs(n_pad, pb):
    return pl.kernel(
        functools.partial(_sc_pairs_body, pb),
        out_type=jax.ShapeDtypeStruct((NW, pb), jnp.float32),
        mesh=_sc_mesh(),
        scratch_types=[
            pltpu.VMEM((n_pad // 128, 128), jnp.float32),
            pltpu.VMEM((n_pad // 128, 128), jnp.float32),
            pltpu.VMEM((pb,), jnp.int32),
            pltpu.VMEM((pb,), jnp.int32),
            pltpu.VMEM((pb,), jnp.float32),
        ],
        compiler_params=pltpu.CompilerParams(needs_layout_passes=False),
        name="sc_link_sigmoid",
    )


def _tc_dense1_body(n, x_ref, aggp_ref, cnt_ref, wl_ref, bl_ref, wr_ref,
                    o_ref):
    cnt = cnt_ref[0, :n, 0] + cnt_ref[1, :n, 0]
    agg = aggp_ref[0, :n, :] + aggp_ref[1, :n, :]
    mean = agg / jnp.maximum(cnt, 1.0)[:, None]
    h = (jnp.dot(mean, wl_ref[...], preferred_element_type=jnp.float32)
         + bl_ref[...][None, :]
         + jnp.dot(x_ref[...], wr_ref[...], preferred_element_type=jnp.float32))
    o_ref[...] = jnp.maximum(h, 0.0)


def _tc_dense2_body(n, h1_ref, aggp_ref, cnt_ref, wl_ref, bl_ref, wr_ref,
                    wps_ref, wpd_ref, bp_ref, u_ref, v_ref):
    cnt = cnt_ref[0, :n, 0] + cnt_ref[1, :n, 0]
    agg = aggp_ref[0, :n, :] + aggp_ref[1, :n, :]
    mean = agg / jnp.maximum(cnt, 1.0)[:, None]
    h2 = (jnp.dot(mean, wl_ref[...], preferred_element_type=jnp.float32)
          + bl_ref[...][None, :]
          + jnp.dot(h1_ref[...], wr_ref[...], preferred_element_type=jnp.float32))
    u_ref[...] = (jnp.dot(h2, wps_ref[...], preferred_element_type=jnp.float32)
                  + bp_ref[...][None, :])
    v_ref[...] = jnp.dot(h2, wpd_ref[...], preferred_element_type=jnp.float32)


def kernel(x, edge_index, pairs, W1_l, b1_l, W1_r, W2_l, b2_l, W2_r, Wp, bp):
    n, d = x.shape
    e = edge_index.shape[1]
    p = pairs.shape[0]

    n_pad = ((n + 1 + 8 * NS - 1) // (8 * NS)) * (8 * NS)
    per_pair = 2 * (((e + NW - 1) // NW + CHUNK - 1) // CHUNK)
    per_pair += per_pair % 2
    k0 = min(per_pair - 2, max(2, 2 * round(per_pair * CORE0_SHARE / 2)))
    k1 = per_pair - k0
    ib = 56
    k_arr = ((max(k0, k1) + ib - 1) // ib) * ib
    e_pad = NS * (k0 + k1) * CHUNK
    pb = ((p + NW - 1) // NW + L - 1) // L * L
    p_pad = pb * NW

    def _split(idx, fill):
        flat = jnp.pad(idx.astype(jnp.int32), (0, e_pad - e),
                       constant_values=fill)
        cut = NS * k0 * CHUNK
        p0_ = flat[:cut].reshape(NS, k0, CHUNK)
        p1_ = flat[cut:].reshape(NS, k1, CHUNK)
        p0_ = jnp.pad(p0_, ((0, 0), (0, k_arr - k0), (0, 0)),
                      constant_values=fill)
        p1_ = jnp.pad(p1_, ((0, 0), (0, k_arr - k1), (0, 0)),
                      constant_values=fill)
        return jnp.concatenate([p0_, p1_], axis=0)

    src = _split(edge_index[0], 0)
    dst = _split(edge_index[1], n)
    k_ev = per_pair // 2
    dst_even = jnp.pad(edge_index[1].astype(jnp.int32),
                       (0, NW * k_ev * CHUNK - e),
                       constant_values=n).reshape(NW, k_ev, CHUNK)
    p0 = jnp.pad(pairs[:, 0].astype(jnp.int32), (0, p_pad - p)).reshape(NW, pb)
    p1 = jnp.pad(pairs[:, 1].astype(jnp.int32), (0, p_pad - p)).reshape(NW, pb)

    zeros_big = jnp.zeros((n_pad, d), jnp.float32)
    ones_big = jnp.ones((CHUNK, d), jnp.float32)
    wp_s = Wp[:d, :]
    wp_d = Wp[d:, :]

    sc_agg = _make_sc_agg(n_pad, k0, k1, ib, d)
    sc_cnt = _make_sc_cnt(n_pad, k_ev, k_ev, d)
    sc_pairs = _make_sc_pairs(n_pad, pb)

    cnt16 = sc_cnt(dst_even, zeros_big, ones_big)
    aggp1 = sc_agg(x, src, dst, zeros_big)

    h1 = pl.pallas_call(
        functools.partial(_tc_dense1_body, n),
        out_shape=jax.ShapeDtypeStruct((n, d), jnp.float32),
    )(x, aggp1, cnt16, W1_l, b1_l, W1_r)

    aggp2 = sc_agg(h1, src, dst, zeros_big)

    u, v = pl.pallas_call(
        functools.partial(_tc_dense2_body, n),
        out_shape=(jax.ShapeDtypeStruct((n, 1), jnp.float32),
                   jax.ShapeDtypeStruct((n, 1), jnp.float32)),
    )(h1, aggp2, cnt16, W2_l, b2_l, W2_r, wp_s, wp_d, bp)

    u_pad = jnp.pad(u.reshape(n), (0, n_pad - n)).reshape(n_pad // 128, 128)
    v_pad = jnp.pad(v.reshape(n), (0, n_pad - n)).reshape(n_pad // 128, 128)
    probs = sc_pairs(u_pad, v_pad, p0, p1)
    return probs.reshape(p_pad)[:p]


# speedup vs baseline: 1.0653x; 1.0653x over previous
import functools

import jax
import jax.numpy as jnp
from jax import lax
from jax.experimental import pallas as pl
from jax.experimental.pallas import tpu as pltpu
from jax.experimental.pallas import tpu_sc as plsc

NC = 2
NS = 16
L = 16
NW = NC * NS

CHUNK = 128

CORE0_SHARE = 0.8125


def _sc_mesh():
    return plsc.VectorSubcoreMesh(
        core_axis_name="c", subcore_axis_name="s", num_cores=NC, num_subcores=NS
    )


def _sc_agg_body(k0, k1, ib, stripe, table_hbm, src_hbm, dst_hbm, zeros_hbm,
                 agg_out, src_v, dst_v, rows0_v, rows1_v, agg_sh,
                 gsem, ssem0, ssem1):
    c = lax.axis_index("c")
    s = lax.axis_index("s")
    w = c * NS + s
    row0 = s * stripe
    k_eff = jnp.where(c == 0, k0, k1)

    pltpu.sync_copy(zeros_hbm.at[pl.ds(row0, stripe)],
                    agg_sh.at[pl.ds(row0, stripe)])
    plsc.subcore_barrier()

    def block(b, carry):
        pltpu.sync_copy(src_hbm.at[w, pl.ds(b * ib, ib)], src_v)
        pltpu.sync_copy(dst_hbm.at[w, pl.ds(b * ib, ib)], dst_v)
        pairs = jnp.clip((k_eff - b * ib) // 2, 0, ib // 2)

        def pair(m, carry2):
            j0 = 2 * m
            j1 = 2 * m + 1
            g0 = 2 * m + b * ib
            g1 = g0 + 1
            pltpu.async_copy(table_hbm.at[src_v.at[j0]], rows0_v,
                             gsem).wait()

            @pl.when(m > 0)
            def _():
                pltpu.make_async_copy(
                    rows1_v, agg_sh.at[dst_v.at[j1 - 2]], ssem1).wait()

            pltpu.async_copy(rows0_v, agg_sh.at[dst_v.at[j0]], ssem0,
                             add=True)
            pltpu.async_copy(table_hbm.at[src_v.at[j1]], rows1_v,
                             gsem).wait()
            pltpu.make_async_copy(rows0_v, agg_sh.at[dst_v.at[j0]],
                                  ssem0).wait()
            pltpu.async_copy(rows1_v, agg_sh.at[dst_v.at[j1]], ssem1,
                             add=True)
            return carry2

        lax.fori_loop(0, pairs, pair, 0)

        @pl.when(pairs > 0)
        def _():
            last = 2 * pairs - 1
            pltpu.make_async_copy(rows1_v, agg_sh.at[dst_v.at[last]],
                                  ssem1).wait()
        return carry

    lax.fori_loop(0, (jnp.maximum(k_eff, 1) + ib - 1) // ib, block, 0)
    plsc.subcore_barrier()

    pltpu.sync_copy(agg_sh.at[pl.ds(row0, stripe)],
                    agg_out.at[c, pl.ds(row0, stripe)])


def _make_sc_agg(n_pad, k0, k1, ib, d):
    stripe = n_pad // NS
    return pl.kernel(
        functools.partial(_sc_agg_body, k0, k1, ib, stripe),
        out_type=jax.ShapeDtypeStruct((NC, n_pad, d), jnp.float32),
        mesh=_sc_mesh(),
        scratch_types=[
            pltpu.VMEM((ib, CHUNK), jnp.int32),
            pltpu.VMEM((ib, CHUNK), jnp.int32),
            pltpu.VMEM((CHUNK, d), jnp.float32),
            pltpu.VMEM((CHUNK, d), jnp.float32),
            pltpu.VMEM_SHARED((n_pad, d), jnp.float32),
            pltpu.SemaphoreType.DMA,
            pltpu.SemaphoreType.DMA,
            pltpu.SemaphoreType.DMA,
        ],
        name="sc_segment_sum",
    )


def _sc_cnt_body(k0, k1, stripe, d, dst_hbm, zeros_hbm, ones_hbm,
                 cnt_out, dst_v, ones_v, cnt_sh):
    c = lax.axis_index("c")
    s = lax.axis_index("s")
    w = c * NS + s
    row0 = s * stripe

    pltpu.sync_copy(dst_hbm.at[w], dst_v)
    pltpu.sync_copy(ones_hbm, ones_v)
    pltpu.sync_copy(zeros_hbm.at[pl.ds(row0, stripe)],
                    cnt_sh.at[pl.ds(row0, stripe)])
    plsc.subcore_barrier()

    def step(j, carry):
        pltpu.sync_copy(ones_v, cnt_sh.at[dst_v.at[j]], add=True)
        return carry

    lax.fori_loop(0, jnp.where(c == 0, k0, k1), step, 0)
    plsc.subcore_barrier()
    pltpu.sync_copy(cnt_sh.at[pl.ds(row0, stripe)],
                    cnt_out.at[c, pl.ds(row0, stripe)])


def _make_sc_cnt(n_pad, k0, k1, d):
    stripe = n_pad // NS
    return pl.kernel(
        functools.partial(_sc_cnt_body, k0, k1, stripe, d),
        out_type=jax.ShapeDtypeStruct((NC, n_pad, d), jnp.float32),
        mesh=_sc_mesh(),
        scratch_types=[
            pltpu.VMEM((max(k0, k1), CHUNK), jnp.int32),
            pltpu.VMEM((CHUNK, d), jnp.float32),
            pltpu.VMEM_SHARED((n_pad, d), jnp.float32),
        ],
        name="sc_degree_count",
    )


def _sc_pairs_body(pb, *refs):
    (u_hbm, v_hbm, p0_hbm, p1_hbm, out_hbm,
     u_v, v_v, p0_v, p1_v, out_v) = refs
    c = lax.axis_index("c")
    s = lax.axis_index("s")
    w = c * NS + s

    pltpu.sync_copy(u_hbm, u_v)
    pltpu.sync_copy(v_hbm, v_v)
    pltpu.sync_copy(p0_hbm.at[w], p0_v)
    pltpu.sync_copy(p1_hbm.at[w], p1_v)

    def step(i, carry):
        i0 = p0_v[pl.ds(i * L, L)]
        i1 = p1_v[pl.ds(i * L, L)]
        a = plsc.load_gather(u_v, [lax.shift_right_logical(i0, 7),
                                   lax.bitwise_and(i0, 127)])
        b = plsc.load_gather(v_v, [lax.shift_right_logical(i1, 7),
                                   lax.bitwise_and(i1, 127)])
        z = a + b
        out_v[pl.ds(i * L, L)] = 1.0 / (1.0 + jnp.exp(-z))
        return carry

    lax.fori_loop(0, pb // L, step, 0)
    pltpu.sync_copy(out_v, out_hbm.at[w])


def _make_sc_pairs(n_pad, pb):
    return pl.kernel(
        functools.partial(_sc_pairs_body, pb),
        out_type=jax.ShapeDtypeStruct((NW, pb), jnp.float32),
        mesh=_sc_mesh(),
        scratch_types=[
            pltpu.VMEM((n_pad // 128, 128), jnp.float32),
            pltpu.VMEM((n_pad // 128, 128), jnp.float32),
            pltpu.VMEM((pb,), jnp.int32),
            pltpu.VMEM((pb,), jnp.int32),
            pltpu.VMEM((pb,), jnp.float32),
        ],
        compiler_params=pltpu.CompilerParams(needs_layout_passes=False),
        name="sc_link_sigmoid",
    )


def _tc_dense1_body(n, x_ref, aggp_ref, cnt_ref, wl_ref, bl_ref, wr_ref,
                    o_ref):
    cnt = cnt_ref[0, :n, 0] + cnt_ref[1, :n, 0]
    agg = aggp_ref[0, :n, :] + aggp_ref[1, :n, :]
    mean = agg / jnp.maximum(cnt, 1.0)[:, None]
    h = (jnp.dot(mean, wl_ref[...], preferred_element_type=jnp.float32)
         + bl_ref[...][None, :]
         + jnp.dot(x_ref[...], wr_ref[...], preferred_element_type=jnp.float32))
    o_ref[...] = jnp.maximum(h, 0.0)


def _tc_dense2_body(n, h1_ref, aggp_ref, cnt_ref, wl_ref, bl_ref, wr_ref,
                    wps_ref, wpd_ref, bp_ref, u_ref, v_ref):
    cnt = cnt_ref[0, :n, 0] + cnt_ref[1, :n, 0]
    agg = aggp_ref[0, :n, :] + aggp_ref[1, :n, :]
    mean = agg / jnp.maximum(cnt, 1.0)[:, None]
    h2 = (jnp.dot(mean, wl_ref[...], preferred_element_type=jnp.float32)
          + bl_ref[...][None, :]
          + jnp.dot(h1_ref[...], wr_ref[...], preferred_element_type=jnp.float32))
    u_ref[...] = (jnp.dot(h2, wps_ref[...], preferred_element_type=jnp.float32)
                  + bp_ref[...][None, :])
    v_ref[...] = jnp.dot(h2, wpd_ref[...], preferred_element_type=jnp.float32)


def kernel(x, edge_index, pairs, W1_l, b1_l, W1_r, W2_l, b2_l, W2_r, Wp, bp):
    n, d = x.shape
    e = edge_index.shape[1]
    p = pairs.shape[0]

    n_pad = ((n + 1 + 8 * NS - 1) // (8 * NS)) * (8 * NS)
    per_pair = 2 * (((e + NW - 1) // NW + CHUNK - 1) // CHUNK)
    per_pair += per_pair % 2
    k0 = min(per_pair - 2, max(2, 2 * round(per_pair * CORE0_SHARE / 2)))
    k1 = per_pair - k0
    ib = 56
    k_arr = ((max(k0, k1) + ib - 1) // ib) * ib
    e_pad = NS * (k0 + k1) * CHUNK
    pb = ((p + NW - 1) // NW + L - 1) // L * L
    p_pad = pb * NW

    def _split(idx, fill):
        flat = jnp.pad(idx.astype(jnp.int32), (0, e_pad - e),
                       constant_values=fill)
        cut = NS * k0 * CHUNK
        p0_ = flat[:cut].reshape(NS, k0, CHUNK)
        p1_ = flat[cut:].reshape(NS, k1, CHUNK)
        p0_ = jnp.pad(p0_, ((0, 0), (0, k_arr - k0), (0, 0)),
                      constant_values=fill)
        p1_ = jnp.pad(p1_, ((0, 0), (0, k_arr - k1), (0, 0)),
                      constant_values=fill)
        return jnp.concatenate([p0_, p1_], axis=0)

    src = _split(edge_index[0], 0)
    dst = _split(edge_index[1], n)
    k_ev = per_pair // 2
    dst_even = jnp.pad(edge_index[1].astype(jnp.int32),
                       (0, NW * k_ev * CHUNK - e),
                       constant_values=n).reshape(NW, k_ev, CHUNK)
    p0 = jnp.pad(pairs[:, 0].astype(jnp.int32), (0, p_pad - p)).reshape(NW, pb)
    p1 = jnp.pad(pairs[:, 1].astype(jnp.int32), (0, p_pad - p)).reshape(NW, pb)

    zeros_big = jnp.zeros((n_pad, d), jnp.float32)
    ones_big = jnp.ones((CHUNK, d), jnp.float32)
    wp_s = Wp[:d, :]
    wp_d = Wp[d:, :]

    sc_agg = _make_sc_agg(n_pad, k0, k1, ib, d)
    sc_cnt = _make_sc_cnt(n_pad, k_ev, k_ev, d)
    sc_pairs = _make_sc_pairs(n_pad, pb)

    cnt16 = sc_cnt(dst_even, zeros_big, ones_big)
    aggp1 = sc_agg(x, src, dst, zeros_big)

    h1 = pl.pallas_call(
        functools.partial(_tc_dense1_body, n),
        out_shape=jax.ShapeDtypeStruct((n, d), jnp.float32),
    )(x, aggp1, cnt16, W1_l, b1_l, W1_r)

    aggp2 = sc_agg(h1, src, dst, zeros_big)

    u, v = pl.pallas_call(
        functools.partial(_tc_dense2_body, n),
        out_shape=(jax.ShapeDtypeStruct((n, 1), jnp.float32),
                   jax.ShapeDtypeStruct((n, 1), jnp.float32)),
    )(h1, aggp2, cnt16, W2_l, b2_l, W2_r, wp_s, wp_d, bp)

    u_pad = jnp.pad(u.reshape(n), (0, n_pad - n)).reshape(n_pad // 128, 128)
    v_pad = jnp.pad(v.reshape(n), (0, n_pad - n)).reshape(n_pad // 128, 128)
    probs = sc_pairs(u_pad, v_pad, p0, p1)
    return probs.reshape(p_pad)[:p]
